# baseline (speedup 1.0000x reference)
import functools

import jax
import jax.numpy as jnp
from jax import lax
from jax.experimental import pallas as pl
from jax.experimental.pallas import tpu as pltpu
from jax.experimental.pallas import tpu_sc as plsc

B = 16384
H = 200
D = 32
BT = 128
NT = B // BT
TILES_PER_W = 4
NR = 4
NB = 2
CW = BT * H


@functools.lru_cache(maxsize=None)
def _make():
  info = plsc.get_sparse_core_info()
  nc, ns = info.num_cores, info.num_subcores
  assert nc * ns * TILES_PER_W == NT
  mesh = plsc.VectorSubcoreMesh(core_axis_name="c", subcore_axis_name="s")

  @functools.partial(
      pl.kernel,
      mesh=mesh,
      out_type=jax.ShapeDtypeStruct((H, D // 8, NT, 8, BT), jnp.float32),
      compiler_params=pltpu.CompilerParams(use_tc_tiling_on_sc=False,
                                           needs_layout_passes=False),
      scratch_types=[
          pltpu.VMEM((CW,), jnp.int32),
          pltpu.VMEM((H, BT), jnp.int32),
          pltpu.VMEM((NR, BT, D), jnp.float32),
          pltpu.VMEM((NB, D // 8, 8, BT), jnp.float32),
          pltpu.SemaphoreType.DMA((NR,)),
          pltpu.SemaphoreType.DMA((NB,)),
      ],
  )
  def gather_kernel(widths_hbm, table_hbm, out_hbm, chunk_v, idx_v, rows_v,
                    blk_v, sem_g, sem_o):
    wid = lax.axis_index("s") * nc + lax.axis_index("c")
    iota = lax.iota(jnp.int32, 16)

    def fire_gather(h, r):
      pltpu.async_copy(table_hbm.at[idx_v.at[h]], rows_v.at[r], sem_g.at[r])

    def wait_gather(r):
      pltpu.make_async_copy(table_hbm.at[idx_v.at[0]], rows_v.at[r],
                            sem_g.at[r]).wait()

    def fire_out(h, t, nb):
      for cg in range(D // 8):
        pltpu.async_copy(blk_v.at[nb, cg], out_hbm.at[h, cg, t],
                         sem_o.at[nb])

    def wait_out(nb):
      for cg in range(D // 8):
        pltpu.make_async_copy(blk_v.at[nb, cg], out_hbm.at[0, cg, 0],
                              sem_o.at[nb]).wait()

    def transpose_unit(rp, nb):
      def tr_body(k0, carry):
        idx_k = k0 * 16 + iota
        for cg in range(D // 8):
          for c in range(8):
            g = plsc.load_gather(rows_v.at[rp],
                                 [idx_k, jnp.full((16,), cg * 8 + c,
                                                  jnp.int32)])
            blk_v[nb, cg, c, pl.ds(k0 * 16, 16)] = g
        return carry

      lax.fori_loop(0, BT // 16, tr_body, 0)

    for j in range(TILES_PER_W):
      t = wid * TILES_PER_W + j
      pltpu.sync_copy(widths_hbm.at[pl.ds(t * CW, CW)], chunk_v)

      def idx_body(h, carry):
        for k0 in range(BT // 16):
          src = (k0 * 16 + iota) * H + h
          idx_v[h, pl.ds(k0 * 16, 16)] = plsc.load_gather(chunk_v, [src])
        return carry

      lax.fori_loop(0, H, idx_body, 0)

      for r in range(NR - 1):
        fire_gather(r, r)

      def h_body(hh, carry):
        for rp in range(NR):
          h = hh * NR + rp

          @pl.when(h + NR - 1 < H)
          def _():
            fire_gather(h + NR - 1, (rp + NR - 1) % NR)

          wait_gather(rp)
          nb = rp % NB
          if j == 0:
            @pl.when(h >= NB)
            def _():
              wait_out(nb)
          else:
            wait_out(nb)
          transpose_unit(rp, nb)
          fire_out(h, t, nb)
        return carry

      lax.fori_loop(0, H // NR, h_body, 0)

    for nb in range(NB):
      wait_out(nb)

  return gather_kernel


def kernel(widths, table):
  flat = widths.reshape(B * H)
  out5 = _make()(flat, table)
  return out5.transpose(2, 4, 0, 1, 3).reshape(B, H, D)


# speedup vs baseline: 1.4643x; 1.4643x over previous
import functools

import jax
import jax.numpy as jnp
from jax import lax
from jax.experimental import pallas as pl
from jax.experimental.pallas import tpu as pltpu
from jax.experimental.pallas import tpu_sc as plsc

B = 16384
H = 200
D = 32
BT = 128
SPAN = 256
NSPAN = 2
NR = 4
NB = 2
CW = BT * H


@functools.lru_cache(maxsize=None)
def _make():
  info = plsc.get_sparse_core_info()
  nc, ns = info.num_cores, info.num_subcores
  assert nc * ns * NSPAN * SPAN == B
  mesh = plsc.VectorSubcoreMesh(core_axis_name="c", subcore_axis_name="s")

  @functools.partial(
      pl.kernel,
      mesh=mesh,
      out_type=jax.ShapeDtypeStruct((H, D // 8, B // BT, 8, BT), jnp.float32),
      compiler_params=pltpu.CompilerParams(use_tc_tiling_on_sc=False,
                                           needs_layout_passes=False),
      scratch_types=[
          pltpu.VMEM((CW,), jnp.int32),
          pltpu.VMEM((H, SPAN), jnp.int32),
          pltpu.VMEM((NR, SPAN, D), jnp.float32),
          pltpu.VMEM((NB, D // 8, SPAN // BT, 8, BT), jnp.float32),
          pltpu.SemaphoreType.DMA((NR,)),
          pltpu.SemaphoreType.DMA((NB,)),
      ],
  )
  def gather_kernel(widths_hbm, table_hbm, out_hbm, chunk_v, idx_v, rows_v,
                    blk_v, sem_g, sem_o):
    wid = lax.axis_index("s") * nc + lax.axis_index("c")
    iota = lax.iota(jnp.int32, 16)
    cvecs = [jnp.full((16,), c, jnp.int32) for c in range(D)]

    def fire_gather(h, r):
      pltpu.async_copy(table_hbm.at[idx_v.at[h]], rows_v.at[r], sem_g.at[r])

    def wait_gather(r):
      pltpu.make_async_copy(table_hbm.at[idx_v.at[0]], rows_v.at[r],
                            sem_g.at[r]).wait()

    def fire_out(h, t0, nb):
      for cg in range(D // 8):
        pltpu.async_copy(blk_v.at[nb, cg],
                         out_hbm.at[h, cg, pl.ds(t0, SPAN // BT)],
                         sem_o.at[nb])

    def wait_out(nb):
      for cg in range(D // 8):
        pltpu.make_async_copy(blk_v.at[nb, cg],
                              out_hbm.at[0, cg, pl.ds(0, SPAN // BT)],
                              sem_o.at[nb]).wait()

    def transpose_unit(rp, nb):
      for bt2 in range(SPAN // BT):
        @plsc.parallel_loop(0, BT // 16)
        def _(k0):
          idx_k = bt2 * BT + k0 * 16 + iota
          for cg in range(D // 8):
            for c in range(8):
              g = plsc.load_gather(rows_v.at[rp], [idx_k, cvecs[cg * 8 + c]])
              blk_v[nb, cg, bt2, c, pl.ds(k0 * 16, 16)] = g

    for j in range(NSPAN):
      b0 = wid * (NSPAN * SPAN) + j * SPAN
      for p in range(SPAN // BT):
        pltpu.sync_copy(widths_hbm.at[pl.ds((b0 + p * BT) * H, CW)], chunk_v)

        @plsc.parallel_loop(0, H)
        def _(h):
          for k0 in range(BT // 16):
            src = (k0 * 16 + iota) * H + h
            idx_v[h, pl.ds(p * BT + k0 * 16, 16)] = (
                plsc.load_gather(chunk_v, [src]))

      for r in range(NR - 1):
        fire_gather(r, r)

      def h_body(hh, carry):
        for rp in range(NR):
          h = hh * NR + rp

          @pl.when(h + NR - 1 < H)
          def _():
            fire_gather(h + NR - 1, (rp + NR - 1) % NR)

          wait_gather(rp)
          nb = rp % NB
          if j == 0:
            @pl.when(h >= NB)
            def _():
              wait_out(nb)
          else:
            wait_out(nb)
          transpose_unit(rp, nb)
          fire_out(h, b0 // BT, nb)
        return carry

      lax.fori_loop(0, H // NR, h_body, 0)

    for nb in range(NB):
      wait_out(nb)

  return gather_kernel


def kernel(widths, table):
  flat = widths.reshape(B * H)
  out5 = _make()(flat, table)
  return out5.transpose(2, 4, 0, 1, 3).reshape(B, H, D)
